# baseline (speedup 1.0000x reference)
import functools

import jax
import jax.numpy as jnp
from jax import lax
from jax.experimental import pallas as pl
from jax.experimental.pallas import tpu as pltpu
from jax.experimental.pallas import tpu_sc as plsc

N_NODES_C = 10000
N_EDGES_C = 160000
HIGH = 128
DL = 640
NTYPES = 20

BN = 2000
BE = 800


def _prep_body(x_ref, na_ref, emb_ref, lw_ref, fc1_ref,
               y_ref, p10_ref, p20_ref, p11_ref, p21_ref):
    x = x_ref[...]
    y_ref[...] = lax.dot_general(x, lw_ref[...], (((1,), (1,)), ((), ())),
                                 preferred_element_type=jnp.float32)
    idx = na_ref[...]
    onehot = (idx == lax.broadcasted_iota(jnp.int32, (1, NTYPES), 1))
    onehot = onehot.astype(jnp.float32)
    emb = emb_ref[...]
    for i, (p1_ref, p2_ref) in enumerate([(p10_ref, p20_ref),
                                          (p11_ref, p21_ref)]):
        w = fc1_ref[i]
        m1 = lax.dot_general(emb, w[:, 0:8], (((1,), (1,)), ((), ())),
                             preferred_element_type=jnp.float32)
        m2 = lax.dot_general(emb, w[:, 8:16], (((1,), (1,)), ((), ())),
                             preferred_element_type=jnp.float32)
        p1_ref[...] = jnp.dot(onehot, m1, preferred_element_type=jnp.float32)
        p2_ref[...] = jnp.dot(onehot, m2, preferred_element_type=jnp.float32)


def _prep(x, node_attr, emb_table, lin_weight, fc1_w):
    n = x.shape[0]
    grid = (n // BN,)
    out = pl.pallas_call(
        _prep_body,
        grid=grid,
        in_specs=[
            pl.BlockSpec((BN, 3), lambda i: (i, 0)),
            pl.BlockSpec((BN, 1), lambda i: (i, 0)),
            pl.BlockSpec((NTYPES, 8), lambda i: (0, 0)),
            pl.BlockSpec((HIGH, 3), lambda i: (0, 0)),
            pl.BlockSpec((2, HIGH, 17), lambda i: (0, 0, 0)),
        ],
        out_specs=[pl.BlockSpec((BN, HIGH), lambda i: (i, 0))] * 5,
        out_shape=[jax.ShapeDtypeStruct((n, HIGH), jnp.float32)] * 5,
    )(x, node_attr, emb_table, lin_weight, fc1_w)
    return out


def _mlp_body(wst_ref, p1s_ref, p2d_ref, ys_ref, yd_ref,
              fc1c_ref, b_ref, wT_ref, mdst_ref, msrc_ref):
    warg = (p1s_ref[...] + p2d_ref[...]
            + wst_ref[...] * fc1c_ref[...] + b_ref[...])
    w_gate = warg * jax.nn.sigmoid(warg)
    ys = ys_ref[...]
    yd = yd_ref[...]
    g = w_gate * (ys - yd)
    a = w_gate * (ys + yd) * 0.5
    dxe = jnp.concatenate([g, a, g * a, g * g, a * a], axis=1)
    wT = wT_ref[...]
    t = jnp.tanh(dxe).astype(jnp.bfloat16)
    z = lax.dot_general(t, wT, (((1,), (0,)), ((), ())),
                        preferred_element_type=jnp.float32)
    z = z - jnp.mean(z, axis=1, keepdims=True)
    z = z / jnp.sqrt(jnp.sum(z * z, axis=1, keepdims=True) + 0.001)
    t2 = jnp.tanh(z).astype(jnp.bfloat16)
    z2 = lax.dot_general(t2, wT, (((1,), (0,)), ((), ())),
                         preferred_element_type=jnp.float32)
    m = jnp.tanh(z2)
    mg = w_gate * m[:, 0:HIGH]
    mave = (0.5 * w_gate) * (m[:, HIGH:2 * HIGH] + m[:, 2 * HIGH:3 * HIGH]
                             + m[:, 3 * HIGH:4 * HIGH] + m[:, 4 * HIGH:])
    mdst_ref[...] = mg + mave
    msrc_ref[...] = mave - mg


def _edge_mlp(wstatic2d, p1s, p2d, ys, yd, fc1c, b_row, wT):
    e = wstatic2d.shape[0]
    grid = (e // BE,)
    row = lambda i: (i, 0)
    fixed = lambda i: (0, 0)
    mdst, msrc = pl.pallas_call(
        _mlp_body,
        grid=grid,
        in_specs=[
            pl.BlockSpec((BE, 1), row),
            pl.BlockSpec((BE, HIGH), row),
            pl.BlockSpec((BE, HIGH), row),
            pl.BlockSpec((BE, HIGH), row),
            pl.BlockSpec((BE, HIGH), row),
            pl.BlockSpec((1, HIGH), fixed),
            pl.BlockSpec((1, HIGH), fixed),
            pl.BlockSpec((DL, DL), fixed),
        ],
        out_specs=[pl.BlockSpec((BE, HIGH), row)] * 2,
        out_shape=[jax.ShapeDtypeStruct((e, HIGH), jnp.float32)] * 2,
    )(wstatic2d, p1s, p2d, ys, yd, fc1c, b_row, wT)
    return mdst, msrc


def _leap_body(i, h_ref, y_ref, yold_ref, pa_ref, pb_ref, out_ref):
    dt = jnp.minimum(h_ref[i] * h_ref[i], 0.1)
    ynew = pa_ref[...] + pb_ref[...]
    out_ref[...] = 2.0 * y_ref[...] - yold_ref[...] - dt * ynew


def _leapfrog(i, h, y, y_old, pa, pb):
    n = y.shape[0]
    grid = (n // BN,)
    row = lambda j: (j, 0)
    return pl.pallas_call(
        functools.partial(_leap_body, i),
        grid=grid,
        in_specs=[
            pl.BlockSpec(memory_space=pltpu.SMEM),
            pl.BlockSpec((BN, HIGH), row),
            pl.BlockSpec((BN, HIGH), row),
            pl.BlockSpec((BN, HIGH), row),
            pl.BlockSpec((BN, HIGH), row),
        ],
        out_specs=pl.BlockSpec((BN, HIGH), row),
        out_shape=jax.ShapeDtypeStruct((n, HIGH), jnp.float32),
    )(h, y, y_old, pa, pb)


def _final_body(i, h_ref, y_ref, yold_ref, pa_ref, pb_ref, lw_ref, out_ref):
    dt = jnp.minimum(h_ref[i] * h_ref[i], 0.1)
    y2 = 2.0 * y_ref[...] - yold_ref[...] - dt * (pa_ref[...] + pb_ref[...])
    out_ref[...] = jnp.dot(y2, lw_ref[...], preferred_element_type=jnp.float32)


def _final(i, h, y, y_old, pa, pb, lin_weight):
    n = y.shape[0]
    grid = (n // BN,)
    row = lambda j: (j, 0)
    return pl.pallas_call(
        functools.partial(_final_body, i),
        grid=grid,
        in_specs=[
            pl.BlockSpec(memory_space=pltpu.SMEM),
            pl.BlockSpec((BN, HIGH), row),
            pl.BlockSpec((BN, HIGH), row),
            pl.BlockSpec((BN, HIGH), row),
            pl.BlockSpec((BN, HIGH), row),
            pl.BlockSpec((HIGH, 3), lambda j: (0, 0)),
        ],
        out_specs=pl.BlockSpec((BN, 3), lambda j: (j, 0)),
        out_shape=jax.ShapeDtypeStruct((n, 3), jnp.float32),
    )(h, y, y_old, pa, pb, lin_weight)


NC = 2
NS = 16
NW = NC * NS
SC_C = 128
N_CHUNK = N_EDGES_C // SC_C
MAX_CH_PER_W = (N_CHUNK + NW - 1) // NW

_sc_mesh = plsc.VectorSubcoreMesh(core_axis_name="c", subcore_axis_name="s")


def _gather_sc_body(y_hbm, p1_hbm, p2_hbm, is_hbm, id_hbm,
                    ys_hbm, yd_hbm, p1s_hbm, p2d_hbm,
                    idx_s, idx_d, buf, sem):
    w = lax.axis_index("s") * NC + lax.axis_index("c")

    def chunk(j, _):
        cid = w + NW * j

        @pl.when(cid < N_CHUNK)
        def _():
            base = cid * SC_C
            pltpu.sync_copy(is_hbm.at[cid], idx_s)
            pltpu.sync_copy(id_hbm.at[cid], idx_d)
            for table, idx, out in ((y_hbm, idx_s, ys_hbm),
                                    (y_hbm, idx_d, yd_hbm),
                                    (p1_hbm, idx_s, p1s_hbm),
                                    (p2_hbm, idx_d, p2d_hbm)):
                pltpu.async_copy(table.at[idx], buf, sem).wait()
                pltpu.sync_copy(buf, out.at[pl.ds(base, SC_C)])
        return None

    lax.fori_loop(0, MAX_CH_PER_W, chunk, None)


def _gather4(y, p1, p2, src2d, dst2d):
    fn = pl.kernel(
        _gather_sc_body,
        out_type=[jax.ShapeDtypeStruct((N_EDGES_C, HIGH), jnp.float32)] * 4,
        mesh=_sc_mesh,
        scratch_types=[
            pltpu.VMEM((SC_C,), jnp.int32),
            pltpu.VMEM((SC_C,), jnp.int32),
            pltpu.VMEM((SC_C, HIGH), jnp.float32),
            pltpu.SemaphoreType.DMA,
        ],
    )
    return fn(y, p1, p2, src2d, dst2d)


_RPT = 624
_TAIL0 = NS * _RPT
_TAIL = N_NODES_C - _TAIL0


def _scatter_sc_body(mdst_hbm, msrc_hbm, is_hbm, id_hbm, zeros_hbm,
                     out_hbm, idx_v, buf, acc):
    c = lax.axis_index("c")
    s = lax.axis_index("s")
    w = s * NC + c
    r0 = s * _RPT
    pltpu.sync_copy(zeros_hbm.at[pl.ds(r0, _RPT)], acc.at[pl.ds(r0, _RPT)])

    @pl.when(s == NS - 1)
    def _():
        pltpu.sync_copy(zeros_hbm.at[pl.ds(_TAIL0, _TAIL)],
                        acc.at[pl.ds(_TAIL0, _TAIL)])

    plsc.subcore_barrier()

    def chunk(j, _):
        cid = w + NW * j

        @pl.when(cid < N_CHUNK)
        def _():
            base = cid * SC_C
            pltpu.sync_copy(id_hbm.at[cid], idx_v)
            pltpu.sync_copy(mdst_hbm.at[pl.ds(base, SC_C)], buf)
            pltpu.sync_copy(buf, acc.at[idx_v], add=True)
            pltpu.sync_copy(is_hbm.at[cid], idx_v)
            pltpu.sync_copy(msrc_hbm.at[pl.ds(base, SC_C)], buf)
            pltpu.sync_copy(buf, acc.at[idx_v], add=True)
        return None

    lax.fori_loop(0, MAX_CH_PER_W, chunk, None)
    plsc.subcore_barrier()
    pltpu.sync_copy(acc.at[pl.ds(r0, _RPT)], out_hbm.at[c, pl.ds(r0, _RPT)])

    @pl.when(s == NS - 1)
    def _():
        pltpu.sync_copy(acc.at[pl.ds(_TAIL0, _TAIL)],
                        out_hbm.at[c, pl.ds(_TAIL0, _TAIL)])


def _scatter2(mdst, msrc, src2d, dst2d, zeros):
    fn = pl.kernel(
        _scatter_sc_body,
        out_type=jax.ShapeDtypeStruct((NC, N_NODES_C, HIGH), jnp.float32),
        mesh=_sc_mesh,
        scratch_types=[
            pltpu.VMEM((SC_C,), jnp.int32),
            pltpu.VMEM((SC_C, HIGH), jnp.float32),
            pltpu.VMEM_SHARED((N_NODES_C, HIGH), jnp.float32),
        ],
    )
    part = fn(mdst, msrc, src2d, dst2d, zeros)
    return part[0], part[1]


def kernel(x, batch, node_attr, edge_src, edge_dst, wstatic, emb_table,
           lin_weight, fc1_w, fc1_b, dl_w, h):
    del batch
    n_nodes = x.shape[0]
    y0, p1_0, p2_0, p1_1, p2_1 = _prep(x, node_attr, emb_table, lin_weight,
                                       fc1_w)
    wst2d = wstatic[:, None]
    src2d = edge_src.reshape(N_CHUNK, SC_C)
    dst2d = edge_dst.reshape(N_CHUNK, SC_C)
    zeros = jnp.zeros((n_nodes, HIGH), jnp.float32)
    ptabs = [(p1_0, p2_0), (p1_1, p2_1)]
    y, y_old = y0, y0
    out = None
    for i in range(2):
        p1, p2 = ptabs[i]
        ys, yd, p1s, p2d = _gather4(y, p1, p2, src2d, dst2d)
        fc1c = fc1_w[i, :, 16].reshape(1, HIGH)
        b_row = fc1_b[i].reshape(1, HIGH)
        wT = dl_w[i].T.astype(jnp.bfloat16)
        mdst, msrc = _edge_mlp(wst2d, p1s, p2d, ys, yd, fc1c, b_row, wT)
        pa, pb = _scatter2(mdst, msrc, src2d, dst2d, zeros)
        if i == 0:
            y, y_old = _leapfrog(i, h, y, y_old, pa, pb), y
        else:
            out = _final(i, h, y, y_old, pa, pb, lin_weight)
    return out


# speedup vs baseline: 1.1510x; 1.1510x over previous
import functools

import jax
import jax.numpy as jnp
from jax import lax
from jax.experimental import pallas as pl
from jax.experimental.pallas import tpu as pltpu
from jax.experimental.pallas import tpu_sc as plsc

N_NODES_C = 10000
N_EDGES_C = 160000
HIGH = 128
DL = 640
NTYPES = 20

BN = 2000
BE = 800


def _pack_pair(y_f32, p_bf16):
    yu = lax.bitcast_convert_type(y_f32.astype(jnp.bfloat16), jnp.int16)
    pu = lax.bitcast_convert_type(p_bf16, jnp.int16)
    lo = yu.astype(jnp.int32) & 0xFFFF
    hi = (pu.astype(jnp.int32) & 0xFFFF) << 16
    return lo | hi


def _unpack_pair(t_i32):
    lo = t_i32.astype(jnp.int16)
    hi = lax.shift_right_logical(t_i32, 16).astype(jnp.int16)
    y = lax.bitcast_convert_type(lo, jnp.bfloat16).astype(jnp.float32)
    p = lax.bitcast_convert_type(hi, jnp.bfloat16).astype(jnp.float32)
    return y, p


def _prep_body(x_ref, na_ref, emb_ref, lw_ref, fc1_ref,
               y_ref, ts0_ref, td0_ref, p11_ref, p21_ref):
    x = x_ref[...]
    y = lax.dot_general(x, lw_ref[...], (((1,), (1,)), ((), ())),
                        preferred_element_type=jnp.float32)
    y_ref[...] = y
    idx = na_ref[...]
    onehot = (idx == lax.broadcasted_iota(jnp.int32, (1, NTYPES), 1))
    onehot = onehot.astype(jnp.float32)
    emb = emb_ref[...]
    ps = []
    for i in range(2):
        w = fc1_ref[i]
        m1 = lax.dot_general(emb, w[:, 0:8], (((1,), (1,)), ((), ())),
                             preferred_element_type=jnp.float32)
        m2 = lax.dot_general(emb, w[:, 8:16], (((1,), (1,)), ((), ())),
                             preferred_element_type=jnp.float32)
        p1 = jnp.dot(onehot, m1, preferred_element_type=jnp.float32)
        p2 = jnp.dot(onehot, m2, preferred_element_type=jnp.float32)
        ps.append((p1, p2))
    ts0_ref[...] = _pack_pair(y, ps[0][0].astype(jnp.bfloat16))
    td0_ref[...] = _pack_pair(y, ps[0][1].astype(jnp.bfloat16))
    p11_ref[...] = ps[1][0].astype(jnp.bfloat16)
    p21_ref[...] = ps[1][1].astype(jnp.bfloat16)


def _prep(x, node_attr, emb_table, lin_weight, fc1_w):
    n = x.shape[0]
    grid = (n // BN,)
    out = pl.pallas_call(
        _prep_body,
        grid=grid,
        in_specs=[
            pl.BlockSpec((BN, 3), lambda i: (i, 0)),
            pl.BlockSpec((BN, 1), lambda i: (i, 0)),
            pl.BlockSpec((NTYPES, 8), lambda i: (0, 0)),
            pl.BlockSpec((HIGH, 3), lambda i: (0, 0)),
            pl.BlockSpec((2, HIGH, 17), lambda i: (0, 0, 0)),
        ],
        out_specs=[pl.BlockSpec((BN, HIGH), lambda i: (i, 0))] * 5,
        out_shape=[jax.ShapeDtypeStruct((n, HIGH), jnp.float32),
                   jax.ShapeDtypeStruct((n, HIGH), jnp.int32),
                   jax.ShapeDtypeStruct((n, HIGH), jnp.int32),
                   jax.ShapeDtypeStruct((n, HIGH), jnp.bfloat16),
                   jax.ShapeDtypeStruct((n, HIGH), jnp.bfloat16)],
    )(x, node_attr, emb_table, lin_weight, fc1_w)
    return out


def _mlp_body(wst_ref, ts_ref, td_ref,
              fc1c_ref, b_ref, wT_ref, mdst_ref, msrc_ref):
    ys, p1s = _unpack_pair(ts_ref[...])
    yd, p2d = _unpack_pair(td_ref[...])
    warg = p1s + p2d + wst_ref[...] * fc1c_ref[...] + b_ref[...]
    w_gate = warg * jax.nn.sigmoid(warg)
    g = w_gate * (ys - yd)
    a = w_gate * (ys + yd) * 0.5
    dxe = jnp.concatenate([g, a, g * a, g * g, a * a], axis=1)
    wT = wT_ref[...]
    t = jnp.tanh(dxe).astype(jnp.bfloat16)
    z = lax.dot_general(t, wT, (((1,), (0,)), ((), ())),
                        preferred_element_type=jnp.float32)
    z = z - jnp.mean(z, axis=1, keepdims=True)
    z = z / jnp.sqrt(jnp.sum(z * z, axis=1, keepdims=True) + 0.001)
    t2 = jnp.tanh(z).astype(jnp.bfloat16)
    z2 = lax.dot_general(t2, wT, (((1,), (0,)), ((), ())),
                         preferred_element_type=jnp.float32)
    m = jnp.tanh(z2)
    mg = w_gate * m[:, 0:HIGH]
    mave = (0.5 * w_gate) * (m[:, HIGH:2 * HIGH] + m[:, 2 * HIGH:3 * HIGH]
                             + m[:, 3 * HIGH:4 * HIGH] + m[:, 4 * HIGH:])
    mdst_ref[...] = mg + mave
    msrc_ref[...] = mave - mg


def _edge_mlp(wstatic2d, ts, td, fc1c, b_row, wT):
    e = wstatic2d.shape[0]
    grid = (e // BE,)
    row = lambda i: (i, 0)
    fixed = lambda i: (0, 0)
    mdst, msrc = pl.pallas_call(
        _mlp_body,
        grid=grid,
        in_specs=[
            pl.BlockSpec((BE, 1), row),
            pl.BlockSpec((BE, HIGH), row),
            pl.BlockSpec((BE, HIGH), row),
            pl.BlockSpec((1, HIGH), fixed),
            pl.BlockSpec((1, HIGH), fixed),
            pl.BlockSpec((DL, DL), fixed),
        ],
        out_specs=[pl.BlockSpec((BE, HIGH), row)] * 2,
        out_shape=[jax.ShapeDtypeStruct((e, HIGH), jnp.float32)] * 2,
    )(wstatic2d, ts, td, fc1c, b_row, wT)
    return mdst, msrc


def _leap_body(i, h_ref, y_ref, yold_ref, pa_ref, pb_ref, p1_ref, p2_ref,
               out_ref, ts_ref, td_ref):
    dt = jnp.minimum(h_ref[i] * h_ref[i], 0.1)
    ynew = pa_ref[...] + pb_ref[...]
    y1 = 2.0 * y_ref[...] - yold_ref[...] - dt * ynew
    out_ref[...] = y1
    ts_ref[...] = _pack_pair(y1, p1_ref[...])
    td_ref[...] = _pack_pair(y1, p2_ref[...])


def _leapfrog(i, h, y, y_old, pa, pb, p1, p2):
    n = y.shape[0]
    grid = (n // BN,)
    row = lambda j: (j, 0)
    return pl.pallas_call(
        functools.partial(_leap_body, i),
        grid=grid,
        in_specs=[
            pl.BlockSpec(memory_space=pltpu.SMEM),
            pl.BlockSpec((BN, HIGH), row),
            pl.BlockSpec((BN, HIGH), row),
            pl.BlockSpec((BN, HIGH), row),
            pl.BlockSpec((BN, HIGH), row),
            pl.BlockSpec((BN, HIGH), row),
            pl.BlockSpec((BN, HIGH), row),
        ],
        out_specs=[pl.BlockSpec((BN, HIGH), row)] * 3,
        out_shape=[jax.ShapeDtypeStruct((n, HIGH), jnp.float32),
                   jax.ShapeDtypeStruct((n, HIGH), jnp.int32),
                   jax.ShapeDtypeStruct((n, HIGH), jnp.int32)],
    )(h, y, y_old, pa, pb, p1, p2)


def _final_body(i, h_ref, y_ref, yold_ref, pa_ref, pb_ref, lw_ref, out_ref):
    dt = jnp.minimum(h_ref[i] * h_ref[i], 0.1)
    y2 = 2.0 * y_ref[...] - yold_ref[...] - dt * (pa_ref[...] + pb_ref[...])
    out_ref[...] = jnp.dot(y2, lw_ref[...], preferred_element_type=jnp.float32)


def _final(i, h, y, y_old, pa, pb, lin_weight):
    n = y.shape[0]
    grid = (n // BN,)
    row = lambda j: (j, 0)
    return pl.pallas_call(
        functools.partial(_final_body, i),
        grid=grid,
        in_specs=[
            pl.BlockSpec(memory_space=pltpu.SMEM),
            pl.BlockSpec((BN, HIGH), row),
            pl.BlockSpec((BN, HIGH), row),
            pl.BlockSpec((BN, HIGH), row),
            pl.BlockSpec((BN, HIGH), row),
            pl.BlockSpec((HIGH, 3), lambda j: (0, 0)),
        ],
        out_specs=pl.BlockSpec((BN, 3), lambda j: (j, 0)),
        out_shape=jax.ShapeDtypeStruct((n, 3), jnp.float32),
    )(h, y, y_old, pa, pb, lin_weight)


NC = 2
NS = 16
NW = NC * NS
SC_C = 128
N_CHUNK = N_EDGES_C // SC_C
MAX_CH_PER_W = (N_CHUNK + NW - 1) // NW

@functools.lru_cache(maxsize=1)
def _sc_mesh():
    return plsc.VectorSubcoreMesh(core_axis_name="c", subcore_axis_name="s",
                                  num_cores=NC, num_subcores=NS)


def _gather_sc_body(ts_hbm, td_hbm, is_hbm, id_hbm,
                    es_hbm, ed_hbm,
                    idx_s, idx_d, buf, sem):
    w = lax.axis_index("s") * NC + lax.axis_index("c")

    def chunk(j, _):
        cid = w + NW * j

        @pl.when(cid < N_CHUNK)
        def _():
            base = cid * SC_C
            pltpu.sync_copy(is_hbm.at[cid], idx_s)
            pltpu.sync_copy(id_hbm.at[cid], idx_d)
            for table, idx, out in ((ts_hbm, idx_s, es_hbm),
                                    (td_hbm, idx_d, ed_hbm)):
                pltpu.async_copy(table.at[idx], buf, sem).wait()
                pltpu.sync_copy(buf, out.at[pl.ds(base, SC_C)])
        return None

    lax.fori_loop(0, MAX_CH_PER_W, chunk, None)


def _gather2(ts, td, src2d, dst2d):
    fn = pl.kernel(
        _gather_sc_body,
        out_type=[jax.ShapeDtypeStruct((N_EDGES_C, HIGH), jnp.int32)] * 2,
        mesh=_sc_mesh(),
        scratch_types=[
            pltpu.VMEM((SC_C,), jnp.int32),
            pltpu.VMEM((SC_C,), jnp.int32),
            pltpu.VMEM((SC_C, HIGH), jnp.int32),
            pltpu.SemaphoreType.DMA,
        ],
    )
    return fn(ts, td, src2d, dst2d)


_RPT = 624
_TAIL0 = NS * _RPT
_TAIL = N_NODES_C - _TAIL0


def _scatter_sc_body(mdst_hbm, msrc_hbm, is_hbm, id_hbm, zeros_hbm,
                     out_hbm, idx_v, buf, acc):
    c = lax.axis_index("c")
    s = lax.axis_index("s")
    w = s * NC + c
    r0 = s * _RPT
    pltpu.sync_copy(zeros_hbm.at[pl.ds(r0, _RPT)], acc.at[pl.ds(r0, _RPT)])

    @pl.when(s == NS - 1)
    def _():
        pltpu.sync_copy(zeros_hbm.at[pl.ds(_TAIL0, _TAIL)],
                        acc.at[pl.ds(_TAIL0, _TAIL)])

    plsc.subcore_barrier()

    def chunk(j, _):
        cid = w + NW * j

        @pl.when(cid < N_CHUNK)
        def _():
            base = cid * SC_C
            pltpu.sync_copy(id_hbm.at[cid], idx_v)
            pltpu.sync_copy(mdst_hbm.at[pl.ds(base, SC_C)], buf)
            pltpu.sync_copy(buf, acc.at[idx_v], add=True)
            pltpu.sync_copy(is_hbm.at[cid], idx_v)
            pltpu.sync_copy(msrc_hbm.at[pl.ds(base, SC_C)], buf)
            pltpu.sync_copy(buf, acc.at[idx_v], add=True)
        return None

    lax.fori_loop(0, MAX_CH_PER_W, chunk, None)
    plsc.subcore_barrier()
    pltpu.sync_copy(acc.at[pl.ds(r0, _RPT)], out_hbm.at[c, pl.ds(r0, _RPT)])

    @pl.when(s == NS - 1)
    def _():
        pltpu.sync_copy(acc.at[pl.ds(_TAIL0, _TAIL)],
                        out_hbm.at[c, pl.ds(_TAIL0, _TAIL)])


def _scatter2(mdst, msrc, src2d, dst2d, zeros):
    fn = pl.kernel(
        _scatter_sc_body,
        out_type=jax.ShapeDtypeStruct((NC, N_NODES_C, HIGH), jnp.float32),
        mesh=_sc_mesh(),
        scratch_types=[
            pltpu.VMEM((SC_C,), jnp.int32),
            pltpu.VMEM((SC_C, HIGH), jnp.float32),
            pltpu.VMEM_SHARED((N_NODES_C, HIGH), jnp.float32),
        ],
    )
    part = fn(mdst, msrc, src2d, dst2d, zeros)
    return part[0], part[1]


def kernel(x, batch, node_attr, edge_src, edge_dst, wstatic, emb_table,
           lin_weight, fc1_w, fc1_b, dl_w, h):
    del batch
    n_nodes = x.shape[0]
    y0, ts_0, td_0, p1_1, p2_1 = _prep(x, node_attr, emb_table,
                                       lin_weight, fc1_w)
    wst2d = wstatic[:, None]
    src2d = edge_src.reshape(N_CHUNK, SC_C)
    dst2d = edge_dst.reshape(N_CHUNK, SC_C)
    zeros = jnp.zeros((n_nodes, HIGH), jnp.float32)
    y, y_old = y0, y0
    ts, td = ts_0, td_0
    out = None
    for i in range(2):
        es, ed = _gather2(ts, td, src2d, dst2d)
        fc1c = fc1_w[i, :, 16].reshape(1, HIGH)
        b_row = fc1_b[i].reshape(1, HIGH)
        wT = dl_w[i].T.astype(jnp.bfloat16)
        mdst, msrc = _edge_mlp(wst2d, es, ed, fc1c, b_row, wT)
        pa, pb = _scatter2(mdst, msrc, src2d, dst2d, zeros)
        if i == 0:
            y1, ts, td = _leapfrog(i, h, y, y_old, pa, pb, p1_1, p2_1)
            y, y_old = y1, y
        else:
            out = _final(i, h, y, y_old, pa, pb, lin_weight)
    return out


# speedup vs baseline: 1.2041x; 1.0461x over previous
import functools

import jax
import jax.numpy as jnp
from jax import lax
from jax.experimental import pallas as pl
from jax.experimental.pallas import tpu as pltpu
from jax.experimental.pallas import tpu_sc as plsc

N_NODES_C = 10000
N_EDGES_C = 160000
HIGH = 128
DL = 640
NTYPES = 20

BN = 2000
BE = 1600


def _pack_pair(y_f32, p_bf16):
    yu = lax.bitcast_convert_type(y_f32.astype(jnp.bfloat16), jnp.int16)
    pu = lax.bitcast_convert_type(p_bf16, jnp.int16)
    lo = yu.astype(jnp.int32) & 0xFFFF
    hi = (pu.astype(jnp.int32) & 0xFFFF) << 16
    return lo | hi


def _unpack_pair(t_i32):
    lo = t_i32.astype(jnp.int16)
    hi = lax.shift_right_logical(t_i32, 16).astype(jnp.int16)
    y = lax.bitcast_convert_type(lo, jnp.bfloat16).astype(jnp.float32)
    p = lax.bitcast_convert_type(hi, jnp.bfloat16).astype(jnp.float32)
    return y, p


def _prep_body(x_ref, na_ref, emb_ref, lw_ref, fc1_ref,
               y_ref, ts0_ref, td0_ref, p11_ref, p21_ref):
    x = x_ref[...]
    y = lax.dot_general(x, lw_ref[...], (((1,), (1,)), ((), ())),
                        preferred_element_type=jnp.float32)
    y_ref[...] = y
    idx = na_ref[...]
    onehot = (idx == lax.broadcasted_iota(jnp.int32, (1, NTYPES), 1))
    onehot = onehot.astype(jnp.float32)
    emb = emb_ref[...]
    ps = []
    for i in range(2):
        w = fc1_ref[i]
        m1 = lax.dot_general(emb, w[:, 0:8], (((1,), (1,)), ((), ())),
                             preferred_element_type=jnp.float32)
        m2 = lax.dot_general(emb, w[:, 8:16], (((1,), (1,)), ((), ())),
                             preferred_element_type=jnp.float32)
        p1 = jnp.dot(onehot, m1, preferred_element_type=jnp.float32)
        p2 = jnp.dot(onehot, m2, preferred_element_type=jnp.float32)
        ps.append((p1, p2))
    ts0_ref[...] = _pack_pair(y, ps[0][0].astype(jnp.bfloat16))
    td0_ref[...] = _pack_pair(y, ps[0][1].astype(jnp.bfloat16))
    p11_ref[...] = ps[1][0].astype(jnp.bfloat16)
    p21_ref[...] = ps[1][1].astype(jnp.bfloat16)


def _prep(x, node_attr, emb_table, lin_weight, fc1_w):
    n = x.shape[0]
    grid = (n // BN,)
    out = pl.pallas_call(
        _prep_body,
        grid=grid,
        in_specs=[
            pl.BlockSpec((BN, 3), lambda i: (i, 0)),
            pl.BlockSpec((BN, 1), lambda i: (i, 0)),
            pl.BlockSpec((NTYPES, 8), lambda i: (0, 0)),
            pl.BlockSpec((HIGH, 3), lambda i: (0, 0)),
            pl.BlockSpec((2, HIGH, 17), lambda i: (0, 0, 0)),
        ],
        out_specs=[pl.BlockSpec((BN, HIGH), lambda i: (i, 0))] * 5,
        out_shape=[jax.ShapeDtypeStruct((n, HIGH), jnp.float32),
                   jax.ShapeDtypeStruct((n, HIGH), jnp.int32),
                   jax.ShapeDtypeStruct((n, HIGH), jnp.int32),
                   jax.ShapeDtypeStruct((n, HIGH), jnp.bfloat16),
                   jax.ShapeDtypeStruct((n, HIGH), jnp.bfloat16)],
    )(x, node_attr, emb_table, lin_weight, fc1_w)
    return out


def _mlp_body(wst_ref, ts_ref, td_ref,
              fc1c_ref, b_ref, wT_ref, mdst_ref, msrc_ref):
    ys, p1s = _unpack_pair(ts_ref[...])
    yd, p2d = _unpack_pair(td_ref[...])
    warg = p1s + p2d + wst_ref[...] * fc1c_ref[...] + b_ref[...]
    w_gate = warg * jax.nn.sigmoid(warg)
    g = w_gate * (ys - yd)
    a = w_gate * (ys + yd) * 0.5
    dxe = jnp.concatenate([g, a, g * a, g * g, a * a], axis=1)
    wT = wT_ref[...]
    dims = (((1,), (0,)), ((), ()))
    t = jnp.tanh(dxe).astype(jnp.bfloat16)
    z = lax.dot_general(t, wT, dims, preferred_element_type=jnp.float32)
    z = z - jnp.mean(z, axis=1, keepdims=True)
    z = z * lax.rsqrt(jnp.sum(z * z, axis=1, keepdims=True) + 0.001)
    t2 = jnp.tanh(z).astype(jnp.bfloat16)
    z2 = lax.dot_general(t2, wT, dims, preferred_element_type=jnp.float32)
    m = jnp.tanh(z2)
    mg = w_gate * m[:, 0:HIGH]
    mave = (0.5 * w_gate) * (m[:, HIGH:2 * HIGH] + m[:, 2 * HIGH:3 * HIGH]
                             + m[:, 3 * HIGH:4 * HIGH] + m[:, 4 * HIGH:])
    mdst_ref[...] = mg + mave
    msrc_ref[...] = mave - mg


def _edge_mlp(wstatic2d, ts, td, fc1c, b_row, wT):
    e = wstatic2d.shape[0]
    grid = (e // BE,)
    row = lambda i: (i, 0)
    fixed = lambda i: (0, 0)
    mdst, msrc = pl.pallas_call(
        _mlp_body,
        grid=grid,
        in_specs=[
            pl.BlockSpec((BE, 1), row),
            pl.BlockSpec((BE, HIGH), row),
            pl.BlockSpec((BE, HIGH), row),
            pl.BlockSpec((1, HIGH), fixed),
            pl.BlockSpec((1, HIGH), fixed),
            pl.BlockSpec((DL, DL), fixed),
        ],
        out_specs=[pl.BlockSpec((BE, HIGH), row)] * 2,
        out_shape=[jax.ShapeDtypeStruct((e, HIGH), jnp.float32)] * 2,
    )(wstatic2d, ts, td, fc1c, b_row, wT)
    return mdst, msrc


def _leap_body(i, h_ref, y_ref, yold_ref, pa_ref, pb_ref, p1_ref, p2_ref,
               out_ref, ts_ref, td_ref):
    dt = jnp.minimum(h_ref[i] * h_ref[i], 0.1)
    ynew = pa_ref[...] + pb_ref[...]
    y1 = 2.0 * y_ref[...] - yold_ref[...] - dt * ynew
    out_ref[...] = y1
    ts_ref[...] = _pack_pair(y1, p1_ref[...])
    td_ref[...] = _pack_pair(y1, p2_ref[...])


def _leapfrog(i, h, y, y_old, pa, pb, p1, p2):
    n = y.shape[0]
    grid = (n // BN,)
    row = lambda j: (j, 0)
    return pl.pallas_call(
        functools.partial(_leap_body, i),
        grid=grid,
        in_specs=[
            pl.BlockSpec(memory_space=pltpu.SMEM),
            pl.BlockSpec((BN, HIGH), row),
            pl.BlockSpec((BN, HIGH), row),
            pl.BlockSpec((BN, HIGH), row),
            pl.BlockSpec((BN, HIGH), row),
            pl.BlockSpec((BN, HIGH), row),
            pl.BlockSpec((BN, HIGH), row),
        ],
        out_specs=[pl.BlockSpec((BN, HIGH), row)] * 3,
        out_shape=[jax.ShapeDtypeStruct((n, HIGH), jnp.float32),
                   jax.ShapeDtypeStruct((n, HIGH), jnp.int32),
                   jax.ShapeDtypeStruct((n, HIGH), jnp.int32)],
    )(h, y, y_old, pa, pb, p1, p2)


def _final_body(i, h_ref, y_ref, yold_ref, pa_ref, pb_ref, lw_ref, out_ref):
    dt = jnp.minimum(h_ref[i] * h_ref[i], 0.1)
    y2 = 2.0 * y_ref[...] - yold_ref[...] - dt * (pa_ref[...] + pb_ref[...])
    out_ref[...] = jnp.dot(y2, lw_ref[...], preferred_element_type=jnp.float32)


def _final(i, h, y, y_old, pa, pb, lin_weight):
    n = y.shape[0]
    grid = (n // BN,)
    row = lambda j: (j, 0)
    return pl.pallas_call(
        functools.partial(_final_body, i),
        grid=grid,
        in_specs=[
            pl.BlockSpec(memory_space=pltpu.SMEM),
            pl.BlockSpec((BN, HIGH), row),
            pl.BlockSpec((BN, HIGH), row),
            pl.BlockSpec((BN, HIGH), row),
            pl.BlockSpec((BN, HIGH), row),
            pl.BlockSpec((HIGH, 3), lambda j: (0, 0)),
        ],
        out_specs=pl.BlockSpec((BN, 3), lambda j: (j, 0)),
        out_shape=jax.ShapeDtypeStruct((n, 3), jnp.float32),
    )(h, y, y_old, pa, pb, lin_weight)


NC = 2
NS = 16
NW = NC * NS
SC_C = 128
N_CHUNK = N_EDGES_C // SC_C
MAX_CH_PER_W = (N_CHUNK + NW - 1) // NW

@functools.lru_cache(maxsize=1)
def _sc_mesh():
    return plsc.VectorSubcoreMesh(core_axis_name="c", subcore_axis_name="s",
                                  num_cores=NC, num_subcores=NS)


def _gather_sc_body(ts_hbm, td_hbm, is_hbm, id_hbm,
                    es_hbm, ed_hbm,
                    idx_s, idx_d, buf, sem):
    w = lax.axis_index("s") * NC + lax.axis_index("c")

    def chunk(j, _):
        cid = w + NW * j

        @pl.when(cid < N_CHUNK)
        def _():
            base = cid * SC_C
            pltpu.sync_copy(is_hbm.at[cid], idx_s)
            pltpu.sync_copy(id_hbm.at[cid], idx_d)
            for table, idx, out in ((ts_hbm, idx_s, es_hbm),
                                    (td_hbm, idx_d, ed_hbm)):
                pltpu.async_copy(table.at[idx], buf, sem).wait()
                pltpu.sync_copy(buf, out.at[pl.ds(base, SC_C)])
        return None

    lax.fori_loop(0, MAX_CH_PER_W, chunk, None)


def _gather2(ts, td, src2d, dst2d):
    fn = pl.kernel(
        _gather_sc_body,
        out_type=[jax.ShapeDtypeStruct((N_EDGES_C, HIGH), jnp.int32)] * 2,
        mesh=_sc_mesh(),
        scratch_types=[
            pltpu.VMEM((SC_C,), jnp.int32),
            pltpu.VMEM((SC_C,), jnp.int32),
            pltpu.VMEM((SC_C, HIGH), jnp.int32),
            pltpu.SemaphoreType.DMA,
        ],
    )
    return fn(ts, td, src2d, dst2d)


_RPT = 624
_TAIL0 = NS * _RPT
_TAIL = N_NODES_C - _TAIL0


def _scatter_sc_body(mdst_hbm, msrc_hbm, is_hbm, id_hbm, zeros_hbm,
                     out_hbm, idx_v, buf, acc):
    c = lax.axis_index("c")
    s = lax.axis_index("s")
    w = s * NC + c
    r0 = s * _RPT
    pltpu.sync_copy(zeros_hbm.at[pl.ds(r0, _RPT)], acc.at[pl.ds(r0, _RPT)])

    @pl.when(s == NS - 1)
    def _():
        pltpu.sync_copy(zeros_hbm.at[pl.ds(_TAIL0, _TAIL)],
                        acc.at[pl.ds(_TAIL0, _TAIL)])

    plsc.subcore_barrier()

    def chunk(j, _):
        cid = w + NW * j

        @pl.when(cid < N_CHUNK)
        def _():
            base = cid * SC_C
            pltpu.sync_copy(id_hbm.at[cid], idx_v)
            pltpu.sync_copy(mdst_hbm.at[pl.ds(base, SC_C)], buf)
            pltpu.sync_copy(buf, acc.at[idx_v], add=True)
            pltpu.sync_copy(is_hbm.at[cid], idx_v)
            pltpu.sync_copy(msrc_hbm.at[pl.ds(base, SC_C)], buf)
            pltpu.sync_copy(buf, acc.at[idx_v], add=True)
        return None

    lax.fori_loop(0, MAX_CH_PER_W, chunk, None)
    plsc.subcore_barrier()
    pltpu.sync_copy(acc.at[pl.ds(r0, _RPT)], out_hbm.at[c, pl.ds(r0, _RPT)])

    @pl.when(s == NS - 1)
    def _():
        pltpu.sync_copy(acc.at[pl.ds(_TAIL0, _TAIL)],
                        out_hbm.at[c, pl.ds(_TAIL0, _TAIL)])


def _scatter2(mdst, msrc, src2d, dst2d, zeros):
    fn = pl.kernel(
        _scatter_sc_body,
        out_type=jax.ShapeDtypeStruct((NC, N_NODES_C, HIGH), jnp.float32),
        mesh=_sc_mesh(),
        scratch_types=[
            pltpu.VMEM((SC_C,), jnp.int32),
            pltpu.VMEM((SC_C, HIGH), jnp.float32),
            pltpu.VMEM_SHARED((N_NODES_C, HIGH), jnp.float32),
        ],
    )
    part = fn(mdst, msrc, src2d, dst2d, zeros)
    return part[0], part[1]


def kernel(x, batch, node_attr, edge_src, edge_dst, wstatic, emb_table,
           lin_weight, fc1_w, fc1_b, dl_w, h):
    del batch
    n_nodes = x.shape[0]
    y0, ts_0, td_0, p1_1, p2_1 = _prep(x, node_attr, emb_table,
                                       lin_weight, fc1_w)
    wst2d = wstatic[:, None]
    src2d = edge_src.reshape(N_CHUNK, SC_C)
    dst2d = edge_dst.reshape(N_CHUNK, SC_C)
    zeros = jnp.zeros((n_nodes, HIGH), jnp.float32)
    y, y_old = y0, y0
    ts, td = ts_0, td_0
    out = None
    for i in range(2):
        es, ed = _gather2(ts, td, src2d, dst2d)
        fc1c = fc1_w[i, :, 16].reshape(1, HIGH)
        b_row = fc1_b[i].reshape(1, HIGH)
        wT = dl_w[i].T.astype(jnp.bfloat16)
        mdst, msrc = _edge_mlp(wst2d, es, ed, fc1c, b_row, wT)
        pa, pb = _scatter2(mdst, msrc, src2d, dst2d, zeros)
        if i == 0:
            y1, ts, td = _leapfrog(i, h, y, y_old, pa, pb, p1_1, p2_1)
            y, y_old = y1, y
        else:
            out = _final(i, h, y, y_old, pa, pb, lin_weight)
    return out


# speedup vs baseline: 1.4521x; 1.2060x over previous
import functools

import jax
import jax.numpy as jnp
from jax import lax
from jax.experimental import pallas as pl
from jax.experimental.pallas import tpu as pltpu
from jax.experimental.pallas import tpu_sc as plsc

N_NODES_C = 10000
N_EDGES_C = 160000
HIGH = 128
DL = 640
NTYPES = 20

BN = 2000
BE = 1600


def _pack_pair(y_f32, p_bf16):
    yu = lax.bitcast_convert_type(y_f32.astype(jnp.bfloat16), jnp.int16)
    pu = lax.bitcast_convert_type(p_bf16, jnp.int16)
    lo = yu.astype(jnp.int32) & 0xFFFF
    hi = (pu.astype(jnp.int32) & 0xFFFF) << 16
    return lo | hi


def _unpack_pair(t_i32):
    lo = t_i32.astype(jnp.int16)
    hi = lax.shift_right_logical(t_i32, 16).astype(jnp.int16)
    y = lax.bitcast_convert_type(lo, jnp.bfloat16).astype(jnp.float32)
    p = lax.bitcast_convert_type(hi, jnp.bfloat16).astype(jnp.float32)
    return y, p


def _prep_body(x_ref, na_ref, emb_ref, lw_ref, fc1_ref,
               y_ref, ts0_ref, td0_ref, p11_ref, p21_ref):
    x = x_ref[...]
    y = lax.dot_general(x, lw_ref[...], (((1,), (1,)), ((), ())),
                        preferred_element_type=jnp.float32)
    y_ref[...] = y
    idx = na_ref[...]
    onehot = (idx == lax.broadcasted_iota(jnp.int32, (1, NTYPES), 1))
    onehot = onehot.astype(jnp.float32)
    emb = emb_ref[...]
    ps = []
    for i in range(2):
        w = fc1_ref[i]
        m1 = lax.dot_general(emb, w[:, 0:8], (((1,), (1,)), ((), ())),
                             preferred_element_type=jnp.float32)
        m2 = lax.dot_general(emb, w[:, 8:16], (((1,), (1,)), ((), ())),
                             preferred_element_type=jnp.float32)
        p1 = jnp.dot(onehot, m1, preferred_element_type=jnp.float32)
        p2 = jnp.dot(onehot, m2, preferred_element_type=jnp.float32)
        ps.append((p1, p2))
    ts0_ref[...] = _pack_pair(y, ps[0][0].astype(jnp.bfloat16))
    td0_ref[...] = _pack_pair(y, ps[0][1].astype(jnp.bfloat16))
    p11_ref[...] = ps[1][0].astype(jnp.bfloat16)
    p21_ref[...] = ps[1][1].astype(jnp.bfloat16)


def _prep(x, node_attr, emb_table, lin_weight, fc1_w):
    n = x.shape[0]
    grid = (n // BN,)
    out = pl.pallas_call(
        _prep_body,
        grid=grid,
        in_specs=[
            pl.BlockSpec((BN, 3), lambda i: (i, 0)),
            pl.BlockSpec((BN, 1), lambda i: (i, 0)),
            pl.BlockSpec((NTYPES, 8), lambda i: (0, 0)),
            pl.BlockSpec((HIGH, 3), lambda i: (0, 0)),
            pl.BlockSpec((2, HIGH, 17), lambda i: (0, 0, 0)),
        ],
        out_specs=[pl.BlockSpec((BN, HIGH), lambda i: (i, 0))] * 5,
        out_shape=[jax.ShapeDtypeStruct((n, HIGH), jnp.float32),
                   jax.ShapeDtypeStruct((n, HIGH), jnp.int32),
                   jax.ShapeDtypeStruct((n, HIGH), jnp.int32),
                   jax.ShapeDtypeStruct((n, HIGH), jnp.bfloat16),
                   jax.ShapeDtypeStruct((n, HIGH), jnp.bfloat16)],
    )(x, node_attr, emb_table, lin_weight, fc1_w)
    return out


def _mlp_body(wst_ref, ts_ref, td_ref,
              fc1c_ref, b_ref, wT_ref, mdst_ref, msrc_ref):
    ys, p1s = _unpack_pair(ts_ref[...])
    yd, p2d = _unpack_pair(td_ref[...])
    warg = p1s + p2d + wst_ref[...] * fc1c_ref[...] + b_ref[...]
    w_gate = warg * jax.nn.sigmoid(warg)
    g = w_gate * (ys - yd)
    a = w_gate * (ys + yd) * 0.5
    dxe = jnp.concatenate([g, a, g * a, g * g, a * a], axis=1)
    wT = wT_ref[...]
    dims = (((1,), (0,)), ((), ()))
    t = jnp.tanh(dxe).astype(jnp.bfloat16)
    z = lax.dot_general(t, wT, dims, preferred_element_type=jnp.float32)
    z = z - jnp.mean(z, axis=1, keepdims=True)
    z = z * lax.rsqrt(jnp.sum(z * z, axis=1, keepdims=True) + 0.001)
    t2 = jnp.tanh(z).astype(jnp.bfloat16)
    z2 = lax.dot_general(t2, wT, dims, preferred_element_type=jnp.float32)
    m = jnp.tanh(z2)
    mg = w_gate * m[:, 0:HIGH]
    mave = (0.5 * w_gate) * (m[:, HIGH:2 * HIGH] + m[:, 2 * HIGH:3 * HIGH]
                             + m[:, 3 * HIGH:4 * HIGH] + m[:, 4 * HIGH:])
    mdst_ref[...] = mg + mave
    msrc_ref[...] = mave - mg


def _edge_mlp(wstatic2d, ts, td, fc1c, b_row, wT):
    e = wstatic2d.shape[0]
    grid = (e // BE,)
    row = lambda i: (i, 0)
    fixed = lambda i: (0, 0)
    mdst, msrc = pl.pallas_call(
        _mlp_body,
        grid=grid,
        in_specs=[
            pl.BlockSpec((BE, 1), row),
            pl.BlockSpec((BE, HIGH), row),
            pl.BlockSpec((BE, HIGH), row),
            pl.BlockSpec((1, HIGH), fixed),
            pl.BlockSpec((1, HIGH), fixed),
            pl.BlockSpec((DL, DL), fixed),
        ],
        out_specs=[pl.BlockSpec((BE, HIGH), row)] * 2,
        out_shape=[jax.ShapeDtypeStruct((e, HIGH), jnp.float32)] * 2,
    )(wstatic2d, ts, td, fc1c, b_row, wT)
    return mdst, msrc


def _leap_body(i, h_ref, y_ref, yold_ref, pa_ref, pb_ref, pc_ref, pd_ref,
               p1_ref, p2_ref, out_ref, ts_ref, td_ref):
    dt = jnp.minimum(h_ref[i] * h_ref[i], 0.1)
    ynew = (pa_ref[...] + pb_ref[...]) + (pc_ref[...] + pd_ref[...])
    y1 = 2.0 * y_ref[...] - yold_ref[...] - dt * ynew
    out_ref[...] = y1
    ts_ref[...] = _pack_pair(y1, p1_ref[...])
    td_ref[...] = _pack_pair(y1, p2_ref[...])


def _leapfrog(i, h, y, y_old, parts, p1, p2):
    n = y.shape[0]
    grid = (n // BN,)
    row = lambda j: (j, 0)
    return pl.pallas_call(
        functools.partial(_leap_body, i),
        grid=grid,
        in_specs=[pl.BlockSpec(memory_space=pltpu.SMEM)]
        + [pl.BlockSpec((BN, HIGH), row)] * 8,
        out_specs=[pl.BlockSpec((BN, HIGH), row)] * 3,
        out_shape=[jax.ShapeDtypeStruct((n, HIGH), jnp.float32),
                   jax.ShapeDtypeStruct((n, HIGH), jnp.int32),
                   jax.ShapeDtypeStruct((n, HIGH), jnp.int32)],
    )(h, y, y_old, *parts, p1, p2)


def _final_body(i, h_ref, y_ref, yold_ref, pa_ref, pb_ref, pc_ref, pd_ref,
                lw_ref, out_ref):
    dt = jnp.minimum(h_ref[i] * h_ref[i], 0.1)
    ynew = (pa_ref[...] + pb_ref[...]) + (pc_ref[...] + pd_ref[...])
    y2 = 2.0 * y_ref[...] - yold_ref[...] - dt * ynew
    out_ref[...] = jnp.dot(y2, lw_ref[...], preferred_element_type=jnp.float32)


def _final(i, h, y, y_old, parts, lin_weight):
    n = y.shape[0]
    grid = (n // BN,)
    row = lambda j: (j, 0)
    return pl.pallas_call(
        functools.partial(_final_body, i),
        grid=grid,
        in_specs=[pl.BlockSpec(memory_space=pltpu.SMEM)]
        + [pl.BlockSpec((BN, HIGH), row)] * 6
        + [pl.BlockSpec((HIGH, 3), lambda j: (0, 0))],
        out_specs=pl.BlockSpec((BN, 3), lambda j: (j, 0)),
        out_shape=jax.ShapeDtypeStruct((n, 3), jnp.float32),
    )(h, y, y_old, *parts, lin_weight)


NC = 2
NS = 16
NW = NC * NS
SC_C = 128
N_CHUNK = N_EDGES_C // SC_C
MAX_CH_PER_W = (N_CHUNK + NW - 1) // NW

@functools.lru_cache(maxsize=1)
def _sc_mesh():
    return plsc.VectorSubcoreMesh(core_axis_name="c", subcore_axis_name="s",
                                  num_cores=NC, num_subcores=NS)


def _gather_sc_body(nchunk, ts_hbm, td_hbm, is_hbm, id_hbm,
                    es_hbm, ed_hbm,
                    idx_s, idx_d, buf, sem):
    w = lax.axis_index("s") * NC + lax.axis_index("c")

    def chunk(j, _):
        cid = w + NW * j

        @pl.when(cid < nchunk)
        def _():
            base = cid * SC_C
            pltpu.sync_copy(is_hbm.at[cid], idx_s)
            pltpu.sync_copy(id_hbm.at[cid], idx_d)
            for table, idx, out in ((ts_hbm, idx_s, es_hbm),
                                    (td_hbm, idx_d, ed_hbm)):
                pltpu.async_copy(table.at[idx], buf, sem).wait()
                pltpu.sync_copy(buf, out.at[pl.ds(base, SC_C)])
        return None

    lax.fori_loop(0, (nchunk + NW - 1) // NW, chunk, None)


def _gather2(ts, td, src2d, dst2d):
    nchunk = src2d.shape[0]
    e = nchunk * SC_C
    fn = pl.kernel(
        functools.partial(_gather_sc_body, nchunk),
        out_type=[jax.ShapeDtypeStruct((e, HIGH), jnp.int32)] * 2,
        mesh=_sc_mesh(),
        scratch_types=[
            pltpu.VMEM((SC_C,), jnp.int32),
            pltpu.VMEM((SC_C,), jnp.int32),
            pltpu.VMEM((SC_C, HIGH), jnp.int32),
            pltpu.SemaphoreType.DMA,
        ],
    )
    return fn(ts, td, src2d, dst2d)


_RPT = 624
_TAIL0 = NS * _RPT
_TAIL = N_NODES_C - _TAIL0


def _scatter_sc_body(nchunk, mdst_hbm, msrc_hbm, is_hbm, id_hbm, zeros_hbm,
                     out_hbm, idx_v, buf, acc):
    c = lax.axis_index("c")
    s = lax.axis_index("s")
    w = s * NC + c
    r0 = s * _RPT
    pltpu.sync_copy(zeros_hbm.at[pl.ds(r0, _RPT)], acc.at[pl.ds(r0, _RPT)])

    @pl.when(s == NS - 1)
    def _():
        pltpu.sync_copy(zeros_hbm.at[pl.ds(_TAIL0, _TAIL)],
                        acc.at[pl.ds(_TAIL0, _TAIL)])

    plsc.subcore_barrier()

    def chunk(j, _):
        cid = w + NW * j

        @pl.when(cid < nchunk)
        def _():
            base = cid * SC_C
            pltpu.sync_copy(id_hbm.at[cid], idx_v)
            pltpu.sync_copy(mdst_hbm.at[pl.ds(base, SC_C)], buf)
            pltpu.sync_copy(buf, acc.at[idx_v], add=True)
            pltpu.sync_copy(is_hbm.at[cid], idx_v)
            pltpu.sync_copy(msrc_hbm.at[pl.ds(base, SC_C)], buf)
            pltpu.sync_copy(buf, acc.at[idx_v], add=True)
        return None

    lax.fori_loop(0, (nchunk + NW - 1) // NW, chunk, None)
    plsc.subcore_barrier()
    pltpu.sync_copy(acc.at[pl.ds(r0, _RPT)], out_hbm.at[c, pl.ds(r0, _RPT)])

    @pl.when(s == NS - 1)
    def _():
        pltpu.sync_copy(acc.at[pl.ds(_TAIL0, _TAIL)],
                        out_hbm.at[c, pl.ds(_TAIL0, _TAIL)])


def _scatter2(mdst, msrc, src2d, dst2d, zeros):
    fn = pl.kernel(
        functools.partial(_scatter_sc_body, src2d.shape[0]),
        out_type=jax.ShapeDtypeStruct((NC, N_NODES_C, HIGH), jnp.float32),
        mesh=_sc_mesh(),
        scratch_types=[
            pltpu.VMEM((SC_C,), jnp.int32),
            pltpu.VMEM((SC_C, HIGH), jnp.float32),
            pltpu.VMEM_SHARED((N_NODES_C, HIGH), jnp.float32),
        ],
    )
    part = fn(mdst, msrc, src2d, dst2d, zeros)
    return part[0], part[1]


def kernel(x, batch, node_attr, edge_src, edge_dst, wstatic, emb_table,
           lin_weight, fc1_w, fc1_b, dl_w, h):
    del batch
    n_nodes = x.shape[0]
    y0, ts_0, td_0, p1_1, p2_1 = _prep(x, node_attr, emb_table,
                                       lin_weight, fc1_w)
    src2d = edge_src.reshape(N_CHUNK, SC_C)
    dst2d = edge_dst.reshape(N_CHUNK, SC_C)
    half = N_CHUNK // 2
    ehalf = half * SC_C
    halves = [(src2d[:half], dst2d[:half], wstatic[:ehalf, None]),
              (src2d[half:], dst2d[half:], wstatic[ehalf:, None])]
    zeros = jnp.zeros((n_nodes, HIGH), jnp.float32)
    y, y_old = y0, y0
    ts, td = ts_0, td_0
    out = None
    for i in range(2):
        fc1c = fc1_w[i, :, 16].reshape(1, HIGH)
        b_row = fc1_b[i].reshape(1, HIGH)
        wT = dl_w[i].T.astype(jnp.bfloat16)
        parts = []
        gathered = [_gather2(ts, td, s2, d2) for s2, d2, _ in halves]
        for (s2, d2, wst), (es, ed) in zip(halves, gathered):
            mdst, msrc = _edge_mlp(wst, es, ed, fc1c, b_row, wT)
            pa, pb = _scatter2(mdst, msrc, s2, d2, zeros)
            parts += [pa, pb]
        if i == 0:
            y1, ts, td = _leapfrog(i, h, y, y_old, parts, p1_1, p2_1)
            y, y_old = y1, y
        else:
            out = _final(i, h, y, y_old, parts, lin_weight)
    return out
